# Initial kernel scaffold; baseline (speedup 1.0000x reference)
import jax
import jax.numpy as jnp
from jax.experimental import pallas as pl


def kernel(sent, table):
    raise NotImplementedError("write your pallas kernel here")


# speedup vs baseline: 1.8314x; 1.8314x over previous
import functools

import jax
import jax.numpy as jnp
from jax import lax
from jax.experimental import pallas as pl
from jax.experimental.pallas import tpu as pltpu
from jax.experimental.pallas import tpu_sc as plsc

GRP = 128


@functools.cache
def _build(n_idx, vocab, d, nw, nc, gpc, nchunk):
    per_w = n_idx // nw
    groups = per_w // GRP
    ch = gpc * GRP

    mesh = plsc.VectorSubcoreMesh(core_axis_name="c", subcore_axis_name="s")

    @functools.partial(
        pl.kernel,
        mesh=mesh,
        compiler_params=pltpu.CompilerParams(use_tc_tiling_on_sc=False),
        out_type=jax.ShapeDtypeStruct((nw, per_w, d), jnp.float32),
        scratch_types=[
            pltpu.VMEM((groups, GRP), jnp.int32),
            pltpu.VMEM((ch, d), jnp.float32),
            pltpu.SemaphoreType.DMA,
        ],
    )
    def emb(table_h, idx_h, out_h, idx_v, rows_v, gsem):
        c = lax.axis_index("c")
        s = lax.axis_index("s")
        wid = s * nc + c
        pltpu.sync_copy(idx_h.at[wid], idx_v)

        def body(i, carry):
            descs = []
            for g in range(gpc):
                grp = i * gpc + g
                descs.append(
                    pltpu.async_copy(
                        table_h.at[idx_v.at[grp]],
                        rows_v.at[pl.ds(g * GRP, GRP)],
                        gsem,
                    )
                )
            for dsc in descs:
                dsc.wait()
            pltpu.sync_copy(rows_v, out_h.at[wid, pl.ds(i * ch, ch)])
            return carry

        lax.fori_loop(0, nchunk, body, 0)

    return emb


def kernel(sent, table):
    b, s = sent.shape
    vocab, d = table.shape
    n = b * s
    nw = 32
    per_w = n // nw
    assert n % (nw * GRP) == 0
    gpc = 4
    nchunk = per_w // (gpc * GRP)
    idx = sent.astype(jnp.int32).reshape(nw, per_w // GRP, GRP)
    out = _build(n, vocab, d, nw, 2, gpc, nchunk)(table, idx)
    return out.reshape(b, s, d)


# speedup vs baseline: 1.8756x; 1.0242x over previous
import functools

import jax
import jax.numpy as jnp
from jax import lax
from jax.experimental import pallas as pl
from jax.experimental.pallas import tpu as pltpu
from jax.experimental.pallas import tpu_sc as plsc

GRP = 128


@functools.cache
def _build(n_idx, vocab, d, nw, nc, gpc, nchunk):
    per_w = n_idx // nw
    groups = per_w // GRP
    ch = gpc * GRP

    mesh = plsc.VectorSubcoreMesh(core_axis_name="c", subcore_axis_name="s")

    assert nchunk % 2 == 0 and nchunk >= 4

    @functools.partial(
        pl.kernel,
        mesh=mesh,
        compiler_params=pltpu.CompilerParams(use_tc_tiling_on_sc=False),
        out_type=jax.ShapeDtypeStruct((nw, per_w, d), jnp.float32),
        scratch_types=[
            pltpu.VMEM((groups, GRP), jnp.int32),
            pltpu.VMEM((2, ch, d), jnp.float32),
            pltpu.SemaphoreType.DMA,
            pltpu.SemaphoreType.DMA,
            pltpu.SemaphoreType.DMA,
            pltpu.SemaphoreType.DMA,
        ],
    )
    def emb(table_h, idx_h, out_h, idx_v, rows_v, gsem0, gsem1, osem0, osem1):
        c = lax.axis_index("c")
        s = lax.axis_index("s")
        wid = s * nc + c
        gsems = (gsem0, gsem1)
        osems = (osem0, osem1)
        pltpu.sync_copy(idx_h.at[wid], idx_v)

        def gather(chunk, b):
            for g in range(gpc):
                pltpu.async_copy(
                    table_h.at[idx_v.at[chunk * gpc + g]],
                    rows_v.at[b, pl.ds(g * GRP, GRP)],
                    gsems[b],
                )

        def gather_wait(b):
            pltpu.make_async_copy(
                out_h.at[wid, pl.ds(0, ch)], rows_v.at[b], gsems[b]
            ).wait()

        def out_start(chunk, b):
            pltpu.async_copy(
                rows_v.at[b], out_h.at[wid, pl.ds(chunk * ch, ch)], osems[b]
            )

        def out_wait(b):
            pltpu.make_async_copy(
                rows_v.at[b], out_h.at[wid, pl.ds(0, ch)], osems[b]
            ).wait()

        gather(0, 0)
        gather(1, 1)

        def body(k, carry):
            chunk = 2 * k
            for b in range(2):
                gather_wait(b)
                out_start(chunk + b, b)
                out_wait(b)
                gather(chunk + 2 + b, b)
            return carry

        lax.fori_loop(0, nchunk // 2 - 1, body, 0)

        for b in range(2):
            gather_wait(b)
            out_start(nchunk - 2 + b, b)
        for b in range(2):
            out_wait(b)

    return emb


def kernel(sent, table):
    b, s = sent.shape
    vocab, d = table.shape
    n = b * s
    nw = 32
    per_w = n // nw
    assert n % (nw * GRP) == 0
    gpc = 4
    nchunk = per_w // (gpc * GRP)
    idx = sent.astype(jnp.int32).reshape(nw, per_w // GRP, GRP)
    out = _build(n, vocab, d, nw, 2, gpc, nchunk)(table, idx)
    return out.reshape(b, s, d)
